# baseline (speedup 1.0000x reference)
import functools

import jax
import jax.numpy as jnp
from jax import lax
from jax.experimental import pallas as pl
from jax.experimental.pallas import tpu as pltpu
from jax.experimental.pallas import tpu_sc as plsc

B = 16384
D = 32
EPS = 1e-5

NC = 2
NS = 16
NW = NC * NS
BPW = B // NW
CHUNK = 128
NCHUNK = BPW // CHUNK

def _sc_gather_body(cid_hbm, pid_hbm, emb_c_hbm, emb_p_hbm, ce_out, pe_out,
                    idx_c, idx_p, rows_c, rows_p, sem):
    wid = lax.axis_index("s") * NC + lax.axis_index("c")
    row_base = wid * NCHUNK
    base = wid * BPW

    pltpu.sync_copy(cid_hbm.at[pl.ds(row_base, NCHUNK)], idx_c)
    pltpu.sync_copy(pid_hbm.at[pl.ds(row_base, NCHUNK)], idx_p)

    copies = []
    for j in range(NCHUNK):
        copies.append(pltpu.async_copy(
            emb_c_hbm.at[idx_c.at[j]], rows_c.at[pl.ds(j * CHUNK, CHUNK)], sem))
        copies.append(pltpu.async_copy(
            emb_p_hbm.at[idx_p.at[j]], rows_p.at[pl.ds(j * CHUNK, CHUNK)], sem))
    for c in copies:
        c.wait()

    pltpu.sync_copy(rows_c, ce_out.at[pl.ds(base, BPW)])
    pltpu.sync_copy(rows_p, pe_out.at[pl.ds(base, BPW)])


@functools.cache
def _sc_gather():
    mesh = plsc.VectorSubcoreMesh(
        core_axis_name="c", subcore_axis_name="s", num_cores=NC, num_subcores=NS
    )
    return pl.kernel(
        _sc_gather_body,
        out_type=(
            jax.ShapeDtypeStruct((B, D), jnp.float32),
            jax.ShapeDtypeStruct((B, D), jnp.float32),
        ),
        mesh=mesh,
        scratch_types=[
            pltpu.VMEM((NCHUNK, CHUNK), jnp.int32),
            pltpu.VMEM((NCHUNK, CHUNK), jnp.int32),
            pltpu.VMEM((BPW, D), jnp.float32),
            pltpu.VMEM((BPW, D), jnp.float32),
            pltpu.SemaphoreType.DMA,
        ],
        compiler_params=pltpu.CompilerParams(use_tc_tiling_on_sc=False),
    )


MLP_BLK = 2048


def _mlp_body(ce_ref, pe_ref, a1c_ref, a1p_ref, c1_ref, a2_ref, c2_ref,
              w3_ref, b3_ref, out_ref):
    h1 = jnp.dot(ce_ref[...], a1c_ref[...], preferred_element_type=jnp.float32)
    h1 = h1 + jnp.dot(pe_ref[...], a1p_ref[...], preferred_element_type=jnp.float32)
    h1 = jnp.maximum(h1 + c1_ref[...], 0.0)
    h2 = jnp.dot(h1, a2_ref[...], preferred_element_type=jnp.float32)
    h2 = jnp.maximum(h2 + c2_ref[...], 0.0)
    o = jnp.sum(h2 * w3_ref[...], axis=1) + b3_ref[0, 0]
    out_ref[...] = 1.0 / (1.0 + jnp.exp(-o))


def _mlp(ce, pe, a1c_t, a1p_t, c1, a2_t, c2, w3, b3):
    grid = (B // MLP_BLK,)
    full = lambda shape: pl.BlockSpec(shape, lambda i: (0, 0))
    return pl.pallas_call(
        _mlp_body,
        grid=grid,
        in_specs=[
            pl.BlockSpec((MLP_BLK, D), lambda i: (i, 0)),
            pl.BlockSpec((MLP_BLK, D), lambda i: (i, 0)),
            full((D, 128)),
            full((D, 128)),
            full((1, 128)),
            full((128, 64)),
            full((1, 64)),
            full((1, 64)),
            full((1, 1)),
        ],
        out_specs=pl.BlockSpec((MLP_BLK,), lambda i: (i,)),
        out_shape=jax.ShapeDtypeStruct((B,), jnp.float32),
    )(ce, pe, a1c_t, a1p_t, c1, a2_t, c2, w3, b3)


def kernel(cliente, producto, emb_c, emb_p, W1, b1, g1, be1, W2, b2, g2, be2,
           W3, b3):
    s1 = g1 * (1.0 / jnp.sqrt(1.0 + EPS))
    a1 = W1 * s1[:, None]
    a1c_t = a1[:, :D].T
    a1p_t = a1[:, D:].T
    c1 = (b1 * s1 + be1).reshape(1, 128)
    s2 = g2 * (1.0 / jnp.sqrt(1.0 + EPS))
    a2_t = (W2 * s2[:, None]).T
    c2 = (b2 * s2 + be2).reshape(1, 64)
    w3 = W3.reshape(1, 64)
    b3v = b3.reshape(1, 1)

    cid = cliente.astype(jnp.int32).reshape(B // CHUNK, CHUNK)
    pid = producto.astype(jnp.int32).reshape(B // CHUNK, CHUNK)

    ce, pe = _sc_gather()(cid, pid, emb_c, emb_p)
    return _mlp(ce, pe, a1c_t, a1p_t, c1, a2_t, c2, w3, b3v)
